# baseline (speedup 1.0000x reference)
import jax
import jax.numpy as jnp
from jax import lax
from jax.experimental import pallas as pl
from jax.experimental.pallas import tpu as pltpu
from jax.experimental.pallas import tpu_sc as plsc

NF = 8
D = 64
V = 35
DW = NF * D
CH = 64
NWORK = 32


def _body(xoff_hbm, wcat_hbm, out_hbm, idx_v, st0, st1, gsem0, gsem1,
          wsem0, wsem1):
    L = idx_v.shape[1]
    nch = L // CH
    wid = lax.axis_index("s") * 2 + lax.axis_index("c")

    pltpu.sync_copy(xoff_hbm.at[wid], idx_v)

    stages = (st0, st1)
    gsems = (gsem0, gsem1)
    wsems = (wsem0, wsem1)

    def step(c, slot):
        stage, gsem, wsem = stages[slot], gsems[slot], wsems[slot]

        @pl.when(c >= 2)
        def _():
            for i in range(NF):
                pltpu.make_async_copy(
                    stage.at[i],
                    out_hbm.at[wid, pl.ds(0, CH), pl.ds(i * D, D)],
                    wsem).wait()

        for i in range(NF):
            pltpu.async_copy(
                wcat_hbm.at[idx_v.at[i, pl.ds(c * CH, CH)]], stage.at[i],
                gsem)
        for i in range(NF):
            pltpu.make_async_copy(
                wcat_hbm.at[idx_v.at[i, pl.ds(0, CH)]], stage.at[i],
                gsem).wait()

        for i in range(NF):
            pltpu.async_copy(
                stage.at[i],
                out_hbm.at[wid, pl.ds(c * CH, CH), pl.ds(i * D, D)],
                wsem)

    def pair(o, _):
        for phase in range(2):
            step(2 * o + phase, phase)
        return 0
    lax.fori_loop(0, nch // 2, pair, 0)

    for slot in range(2):
        for i in range(NF):
            pltpu.make_async_copy(
                stages[slot].at[i],
                out_hbm.at[wid, pl.ds(0, CH), pl.ds(i * D, D)],
                wsems[slot]).wait()


def kernel(x, W0, W1, W2, W3, W4, W5, W6, W7):
    B, nf, L = x.shape
    assert nf == NF and B == NWORK and L % (2 * CH) == 0
    tables = (W0, W1, W2, W3, W4, W5, W6, W7)
    wcat = jnp.concatenate([w[:V] for w in tables], axis=0)
    xoff = x.astype(jnp.int32) + (V * jnp.arange(NF, dtype=jnp.int32))[None, :, None]

    mesh = plsc.VectorSubcoreMesh(core_axis_name="c", subcore_axis_name="s")
    f = pl.kernel(
        _body,
        compiler_params=pltpu.CompilerParams(
            use_tc_tiling_on_sc=False, needs_layout_passes=False),
        out_type=jax.ShapeDtypeStruct((B, L, DW), jnp.float32),
        mesh=mesh,
        scratch_types=[
            pltpu.VMEM((NF, L), jnp.int32),
            pltpu.VMEM((NF, CH, D), jnp.float32),
            pltpu.VMEM((NF, CH, D), jnp.float32),
            pltpu.SemaphoreType.DMA,
            pltpu.SemaphoreType.DMA,
            pltpu.SemaphoreType.DMA,
            pltpu.SemaphoreType.DMA,
        ],
    )
    return f(xoff, wcat)


# speedup vs baseline: 2.1202x; 2.1202x over previous
import jax
import jax.numpy as jnp
from jax import lax
from jax.experimental import pallas as pl
from jax.experimental.pallas import tpu as pltpu
from jax.experimental.pallas import tpu_sc as plsc

NF = 8
D = 64
V = 35
DW = NF * D
CH = 64
NWORK = 32


def _body(xoff_hbm, wcat_hbm, out_hbm, idx_v, tbl_sh, st0, st1, gsem0, gsem1,
          wsem0, wsem1):
    L = idx_v.shape[1]
    nch = L // CH
    wid = lax.axis_index("s") * 2 + lax.axis_index("c")

    @pl.when(lax.axis_index("s") == 0)
    def _():
        pltpu.sync_copy(wcat_hbm, tbl_sh)
    pltpu.sync_copy(xoff_hbm.at[wid], idx_v)
    plsc.subcore_barrier()

    stages = (st0, st1)
    gsems = (gsem0, gsem1)
    wsems = (wsem0, wsem1)

    def step(c, slot):
        stage, gsem, wsem = stages[slot], gsems[slot], wsems[slot]

        @pl.when(c >= 2)
        def _():
            for i in range(NF):
                pltpu.make_async_copy(
                    stage.at[i],
                    out_hbm.at[wid, pl.ds(0, CH), pl.ds(i * D, D)],
                    wsem).wait()

        for i in range(NF):
            pltpu.async_copy(
                tbl_sh.at[idx_v.at[i, pl.ds(c * CH, CH)]], stage.at[i],
                gsem)
        for i in range(NF):
            pltpu.make_async_copy(
                tbl_sh.at[idx_v.at[i, pl.ds(0, CH)]], stage.at[i],
                gsem).wait()

        for i in range(NF):
            pltpu.async_copy(
                stage.at[i],
                out_hbm.at[wid, pl.ds(c * CH, CH), pl.ds(i * D, D)],
                wsem)

    def pair(o, _):
        for phase in range(2):
            step(2 * o + phase, phase)
        return 0
    lax.fori_loop(0, nch // 2, pair, 0)

    for slot in range(2):
        for i in range(NF):
            pltpu.make_async_copy(
                stages[slot].at[i],
                out_hbm.at[wid, pl.ds(0, CH), pl.ds(i * D, D)],
                wsems[slot]).wait()


def kernel(x, W0, W1, W2, W3, W4, W5, W6, W7):
    B, nf, L = x.shape
    assert nf == NF and B == NWORK and L % (2 * CH) == 0
    tables = (W0, W1, W2, W3, W4, W5, W6, W7)
    wcat = jnp.concatenate([w[:V] for w in tables], axis=0)
    xoff = x.astype(jnp.int32) + (V * jnp.arange(NF, dtype=jnp.int32))[None, :, None]

    mesh = plsc.VectorSubcoreMesh(core_axis_name="c", subcore_axis_name="s")
    f = pl.kernel(
        _body,
        compiler_params=pltpu.CompilerParams(
            use_tc_tiling_on_sc=False, needs_layout_passes=False),
        out_type=jax.ShapeDtypeStruct((B, L, DW), jnp.float32),
        mesh=mesh,
        scratch_types=[
            pltpu.VMEM((NF, L), jnp.int32),
            pltpu.VMEM_SHARED((NF * V, D), jnp.float32),
            pltpu.VMEM((NF, CH, D), jnp.float32),
            pltpu.VMEM((NF, CH, D), jnp.float32),
            pltpu.SemaphoreType.DMA,
            pltpu.SemaphoreType.DMA,
            pltpu.SemaphoreType.DMA,
            pltpu.SemaphoreType.DMA,
        ],
    )
    return f(xoff, wcat)
